# baseline (speedup 1.0000x reference)
import functools

import jax
import jax.numpy as jnp
from jax import lax
from jax.experimental import pallas as pl
from jax.experimental.pallas import tpu as pltpu
from jax.experimental.pallas import tpu_sc as plsc

_LANES = 16
_NC, _NS = 2, 16


def _sc_copy_scatter(x, idx, val):
    rows, cols = x.shape
    n = idx.shape[0]
    nw = _NC * _NS
    rpw = rows // nw
    mesh = plsc.VectorSubcoreMesh(core_axis_name="c", subcore_axis_name="s")

    @functools.partial(
        pl.kernel,
        out_type=jax.ShapeDtypeStruct((rows, cols), x.dtype),
        mesh=mesh,
        scratch_types=[
            pltpu.VMEM((cols,), x.dtype),
            pltpu.VMEM((n,), jnp.int32),
            pltpu.VMEM((n,), x.dtype),
            pltpu.SemaphoreType.DMA,
            pltpu.SemaphoreType.DMA,
        ],
        compiler_params=pltpu.CompilerParams(needs_layout_passes=False),
    )
    def k(x_hbm, idx_hbm, val_hbm, o_hbm, row_v, idx_v, val_v, sem_b, sem_r):
        wid = lax.axis_index("s") * _NC + lax.axis_index("c")
        base = wid * rpw
        blk = pltpu.make_async_copy(
            x_hbm.at[pl.ds(base, rpw)], o_hbm.at[pl.ds(base, rpw)], sem_b
        )
        blk.start()

        @pl.when(wid == 0)
        def _():
            pltpu.async_copy(x_hbm.at[0], row_v, sem_r).wait()
            pltpu.sync_copy(idx_hbm, idx_v)
            pltpu.sync_copy(val_hbm, val_v)
            for j in range(n // _LANES):
                iv = idx_v[pl.ds(j * _LANES, _LANES)]
                vv = val_v[pl.ds(j * _LANES, _LANES)]
                plsc.store_scatter(row_v, [iv], vv)

        blk.wait()

        @pl.when(wid == 0)
        def _():
            pltpu.async_copy(row_v, o_hbm.at[0], sem_r).wait()

    return k(x, idx, val)


def kernel(x, changed_genes, change_values):
    idx = changed_genes.astype(jnp.int32)
    n = idx.shape[0]
    pad = (-n) % _LANES
    if pad:
        idx = jnp.concatenate([idx, jnp.broadcast_to(idx[-1:], (pad,))])
        change_values = jnp.concatenate(
            [change_values, jnp.broadcast_to(change_values[-1:], (pad,))]
        )
    return _sc_copy_scatter(x, idx, change_values)
